# Initial kernel scaffold; baseline (speedup 1.0000x reference)
import jax
import jax.numpy as jnp
from jax.experimental import pallas as pl


def kernel(outputs, W, Wh):
    raise NotImplementedError("write your pallas kernel here")


# speedup vs baseline: 11.5173x; 11.5173x over previous
import functools

import jax
import jax.numpy as jnp
from jax import lax
from jax.experimental import pallas as pl
from jax.experimental.pallas import tpu as pltpu
from jax.experimental.pallas import tpu_sc as plsc

N = 4096
M = 62
D = 128
C = 64
NEG = -1e9

CH_S0, CH_S63, CH_DC, CH_G63, CH_N, CH_G0 = 0, 64, 128, 192, 256, 320

R_BLK = 512


def _comp_body(ue_ref, w_ref, src_ref, dst_ref, rd_ref, sh_ref):
    f32 = jnp.float32
    Ue = ue_ref[:]
    W = w_ref[:]
    src = src_ref[:]
    dst = dst_ref[:]

    def mm(a, b):
        return lax.dot_general(a, b, (((1,), (0,)), ((), ())),
                               preferred_element_type=f32)

    def mmt(a, b):
        return lax.dot_general(a, b, (((1,), (1,)), ((), ())),
                               preferred_element_type=f32)

    QUe = mm(Ue, W)
    Qsrc = mm(src, W)
    Qdst = mm(dst, W)

    col = lax.broadcasted_iota(jnp.int32, (R_BLK, C), 1)
    is0 = col == 0
    is63 = col == C - 1

    ndst = jnp.sum(dst * dst, axis=1, keepdims=True)
    nsrc = jnp.sum(src * src, axis=1, keepdims=True)

    S0 = mmt(Qsrc, Ue)
    S63 = mmt(Qdst, Ue)
    S63 = jnp.where(is63, jnp.sum(Qdst * dst, axis=1, keepdims=True), S63)
    DC = mmt(dst, QUe)
    G63 = mmt(dst, Ue)
    G63 = jnp.where(is63, ndst, G63)
    G0 = mmt(src, Ue)
    nU = jnp.sum(Ue * Ue, axis=1)
    NCH = jnp.broadcast_to(nU[None, :], (R_BLK, C))
    NCH = jnp.where(is0, nsrc, NCH)
    NCH = jnp.where(is63, ndst, NCH)

    rd_ref[:] = jnp.concatenate([S0, S63, DC, G63, NCH, G0], axis=1)
    sh_ref[:] = jnp.concatenate([mmt(QUe, Ue), mmt(Ue, Ue)], axis=0)


_components = functools.partial(
    pl.pallas_call,
    _comp_body,
    grid=(N // R_BLK,),
    in_specs=[
        pl.BlockSpec((C, D), lambda i: (0, 0)),
        pl.BlockSpec((D, D), lambda i: (0, 0)),
        pl.BlockSpec((R_BLK, D), lambda i: (i, 0)),
        pl.BlockSpec((R_BLK, D), lambda i: (i, 0)),
    ],
    out_specs=[
        pl.BlockSpec((R_BLK, 6 * C), lambda i: (i, 0)),
        pl.BlockSpec((2 * C, C), lambda i: (0, 0)),
    ],
    out_shape=[
        jax.ShapeDtypeStruct((N, 6 * C), jnp.float32),
        jax.ShapeDtypeStruct((2 * C, C), jnp.float32),
    ],
)


NUM_WORKERS = 32
ROWS_PER_W = N // NUM_WORKERS
GROUPS = ROWS_PER_W // 16

def _decode_body(rd_hbm, sh_hbm, out_hbm, rd_v, sh_v, out_v):
    wid = lax.axis_index("s") * 2 + lax.axis_index("c")
    base = wid * ROWS_PER_W
    pltpu.sync_copy(rd_hbm.at[pl.ds(base, ROWS_PER_W)], rd_v)
    pltpu.sync_copy(sh_hbm, sh_v)

    lanes = lax.iota(jnp.int32, 16)
    zero = jnp.zeros((16,), jnp.int32)
    one = jnp.ones((16,), jnp.int32)
    negv = jnp.full((16,), NEG, jnp.float32)
    ninf = jnp.full((16,), -jnp.inf, jnp.float32)

    for g in range(GROUPS):
        rvec = lanes + (g * 16)

        def gath(chbase, colv):
            return plsc.load_gather(rd_v, [rvec, colv + chbase])

        def s0_body(s, carry):
            best, besti = carry
            sv = zero + s
            v = gath(CH_S0, sv)
            gt = v > best
            return jnp.where(gt, v, best), jnp.where(gt, sv, besti)

        _, sp = lax.fori_loop(1, 63, s0_body, (ninf, zero), unroll=8)
        nn = gath(CH_N, sp)
        d2 = jnp.maximum(gath(CH_N, zero) + nn - 2.0 * gath(CH_G0, sp), 0.0)
        maxd2 = d2
        masklo = jnp.where(sp < 32, one << (sp & 31), zero)
        maskhi = jnp.where(sp >= 32, one << ((sp - 32) & 31), zero)

        def step(_, carry):
            j, ncur, mlo, mhi, md2 = carry
            isD = j == C - 1

            def inner(s, c):
                best, besti = c
                sv = zero + s
                vU = plsc.load_gather(sh_v, [j, sv])
                vD = gath(CH_S63, sv)
                v = jnp.where(isD, vD, vU)
                word = jnp.where(sv < 32, mlo, mhi)
                dead = ((word >> (sv & 31)) & 1) > 0
                cand = jnp.where(dead, negv, v)
                gt = cand > best
                return jnp.where(gt, cand, best), jnp.where(gt, sv, besti)

            best, besti = lax.fori_loop(1, 63, inner, (ninf, zero), unroll=8)
            v63 = jnp.where(isD, gath(CH_S63, zero + 63), gath(CH_DC, j))
            gt = v63 > best
            sp = jnp.where(gt, zero + 63, besti)
            nn = gath(CH_N, sp)
            gU = plsc.load_gather(sh_v, [j + C, sp])
            gv = jnp.where(isD, gath(CH_G63, sp),
                           jnp.where(sp == C - 1, gath(CH_G63, j), gU))
            d2 = jnp.maximum(ncur + nn - 2.0 * gv, 0.0)
            md2 = jnp.maximum(md2, d2)
            mlo = mlo | jnp.where(sp < 32, one << (sp & 31), zero)
            mhi = mhi | jnp.where((sp >= 32) & (sp < 63),
                                  one << ((sp - 32) & 31), zero)
            return sp, nn, mlo, mhi, md2

        carry = (sp, nn, masklo, maskhi, maxd2)
        *_, maxd2 = lax.fori_loop(1, C, step, carry)
        out_v[pl.ds(g * 16, 16)] = maxd2

    pltpu.sync_copy(out_v, out_hbm.at[pl.ds(base, ROWS_PER_W)])


@functools.cache
def _decode():
    mesh = plsc.VectorSubcoreMesh(core_axis_name="c", subcore_axis_name="s",
                                  num_cores=2, num_subcores=16)
    return pl.kernel(
        _decode_body,
        out_type=jax.ShapeDtypeStruct((N,), jnp.float32),
        mesh=mesh,
        scratch_types=[
            pltpu.VMEM((ROWS_PER_W, 6 * C), jnp.float32),
            pltpu.VMEM((2 * C, C), jnp.float32),
            pltpu.VMEM((ROWS_PER_W,), jnp.float32),
        ],
        compiler_params=pltpu.CompilerParams(use_tc_tiling_on_sc=False,
                                             needs_layout_passes=False),
    )


def _final_body(x_ref, o_ref):
    o_ref[0, 0] = jnp.sum(jnp.sqrt(x_ref[:])) * (1.0 / N)


_finalize = functools.partial(
    pl.pallas_call,
    _final_body,
    in_specs=[pl.BlockSpec((NUM_WORKERS, ROWS_PER_W), lambda: (0, 0))],
    out_specs=pl.BlockSpec(memory_space=pltpu.SMEM),
    out_shape=jax.ShapeDtypeStruct((1, 1), jnp.float32),
)


def kernel(outputs, W, Wh):
    del Wh
    src = outputs[:N]
    dst = outputs[N:2 * N]
    U = outputs[2 * N:]
    Ue = jnp.zeros((C, D), jnp.float32).at[1:1 + M].set(U)
    rd, sh = _components()(Ue, W, src, dst)
    maxd2 = _decode()(rd, sh)
    res = _finalize()(maxd2.reshape(NUM_WORKERS, ROWS_PER_W))
    return res[0, 0]


# speedup vs baseline: 27.9086x; 2.4232x over previous
import functools

import jax
import jax.numpy as jnp
from jax import lax
from jax.experimental import pallas as pl
from jax.experimental.pallas import tpu as pltpu
from jax.experimental.pallas import tpu_sc as plsc

N = 4096
M = 62
D = 128
C = 64
NEG = -1e9

CH_S0, CH_S63, CH_DC, CH_G63, CH_N, CH_G0 = 0, 64, 128, 192, 256, 320

R_BLK = 512


def _comp_body(ue_ref, w_ref, src_ref, dst_ref, rd_ref, sh_ref):
    f32 = jnp.float32
    Ue = ue_ref[:]
    W = w_ref[:]
    src = src_ref[:]
    dst = dst_ref[:]

    def mm(a, b):
        return lax.dot_general(a, b, (((1,), (0,)), ((), ())),
                               preferred_element_type=f32)

    def mmt(a, b):
        return lax.dot_general(a, b, (((1,), (1,)), ((), ())),
                               preferred_element_type=f32)

    QUe = mm(Ue, W)
    Qsrc = mm(src, W)
    Qdst = mm(dst, W)

    col = lax.broadcasted_iota(jnp.int32, (R_BLK, C), 1)
    is0 = col == 0
    is63 = col == C - 1

    ndst = jnp.sum(dst * dst, axis=1, keepdims=True)
    nsrc = jnp.sum(src * src, axis=1, keepdims=True)

    S0 = mmt(Qsrc, Ue)
    S63 = mmt(Qdst, Ue)
    S63 = jnp.where(is63, jnp.sum(Qdst * dst, axis=1, keepdims=True), S63)
    DC = mmt(dst, QUe)
    G63 = mmt(dst, Ue)
    G63 = jnp.where(is63, ndst, G63)
    G0 = mmt(src, Ue)
    nU = jnp.sum(Ue * Ue, axis=1)
    NCH = jnp.broadcast_to(nU[None, :], (R_BLK, C))
    NCH = jnp.where(is0, nsrc, NCH)
    NCH = jnp.where(is63, ndst, NCH)

    rd_ref[:] = jnp.concatenate([S0, S63, DC, G63, NCH, G0], axis=1)
    sh_ref[:] = jnp.concatenate([mmt(QUe, Ue), mmt(Ue, Ue)], axis=0)


_components = functools.partial(
    pl.pallas_call,
    _comp_body,
    grid=(N // R_BLK,),
    in_specs=[
        pl.BlockSpec((C, D), lambda i: (0, 0)),
        pl.BlockSpec((D, D), lambda i: (0, 0)),
        pl.BlockSpec((R_BLK, D), lambda i: (i, 0)),
        pl.BlockSpec((R_BLK, D), lambda i: (i, 0)),
    ],
    out_specs=[
        pl.BlockSpec((R_BLK, 6 * C), lambda i: (i, 0)),
        pl.BlockSpec((2 * C, C), lambda i: (0, 0)),
    ],
    out_shape=[
        jax.ShapeDtypeStruct((N, 6 * C), jnp.float32),
        jax.ShapeDtypeStruct((2 * C, C), jnp.float32),
    ],
)


NUM_WORKERS = 32
ROWS_PER_W = N // NUM_WORKERS
GROUPS = ROWS_PER_W // 16
RD_STRIDE = 6 * C + 1
SH_STRIDE = C + 1

def _decode_body(rd_hbm, sh_hbm, out_hbm, rd_v, sh_v, out_v):
    wid = lax.axis_index("s") * 2 + lax.axis_index("c")
    base = wid * ROWS_PER_W
    pltpu.sync_copy(rd_hbm.at[pl.ds(base, ROWS_PER_W)],
                    rd_v.at[:, pl.ds(0, 6 * C)])
    pltpu.sync_copy(sh_hbm, sh_v.at[:, pl.ds(0, C)])

    lanes = lax.iota(jnp.int32, 16)
    zero = jnp.zeros((16,), jnp.int32)
    one = jnp.ones((16,), jnp.int32)
    negv = jnp.full((16,), NEG, jnp.float32)
    ninf = jnp.full((16,), -jnp.inf, jnp.float32)

    for g in range(GROUPS):
        rvec = lanes + (g * 16)

        def gath(chbase, colv):
            return plsc.load_gather(rd_v, [rvec, colv + chbase])

        def s0_body(s, carry):
            best, besti = carry
            sv = zero + s
            v = gath(CH_S0, sv)
            gt = v > best
            return jnp.where(gt, v, best), jnp.where(gt, sv, besti)

        _, sp = lax.fori_loop(1, 63, s0_body, (ninf, zero), unroll=8)
        nn = gath(CH_N, sp)
        d2 = jnp.maximum(gath(CH_N, zero) + nn - 2.0 * gath(CH_G0, sp), 0.0)
        maxd2 = d2
        masklo = jnp.where(sp < 32, one << (sp & 31), zero)
        maskhi = jnp.where(sp >= 32, one << ((sp - 32) & 31), zero)

        def step(_, carry):
            j, ncur, mlo, mhi, md2 = carry
            isD = j == C - 1

            def inner(s, c):
                best, besti = c
                sv = zero + s
                vU = plsc.load_gather(sh_v, [j, sv])
                vD = gath(CH_S63, sv)
                v = jnp.where(isD, vD, vU)
                word = jnp.where(sv < 32, mlo, mhi)
                dead = ((word >> (sv & 31)) & 1) > 0
                cand = jnp.where(dead, negv, v)
                gt = cand > best
                return jnp.where(gt, cand, best), jnp.where(gt, sv, besti)

            best, besti = lax.fori_loop(1, 63, inner, (ninf, zero), unroll=8)
            v63 = jnp.where(isD, gath(CH_S63, zero + 63), gath(CH_DC, j))
            gt = v63 > best
            sp = jnp.where(gt, zero + 63, besti)
            nn = gath(CH_N, sp)
            gU = plsc.load_gather(sh_v, [j + C, sp])
            gv = jnp.where(isD, gath(CH_G63, sp),
                           jnp.where(sp == C - 1, gath(CH_G63, j), gU))
            d2 = jnp.maximum(ncur + nn - 2.0 * gv, 0.0)
            md2 = jnp.maximum(md2, d2)
            mlo = mlo | jnp.where(sp < 32, one << (sp & 31), zero)
            mhi = mhi | jnp.where((sp >= 32) & (sp < 63),
                                  one << ((sp - 32) & 31), zero)
            return sp, nn, mlo, mhi, md2

        carry = (sp, nn, masklo, maskhi, maxd2)
        *_, maxd2 = lax.fori_loop(1, C, step, carry)
        out_v[pl.ds(g * 16, 16)] = maxd2

    pltpu.sync_copy(out_v, out_hbm.at[pl.ds(base, ROWS_PER_W)])


@functools.cache
def _decode():
    mesh = plsc.VectorSubcoreMesh(core_axis_name="c", subcore_axis_name="s",
                                  num_cores=2, num_subcores=16)
    return pl.kernel(
        _decode_body,
        out_type=jax.ShapeDtypeStruct((N,), jnp.float32),
        mesh=mesh,
        scratch_types=[
            pltpu.VMEM((ROWS_PER_W, RD_STRIDE), jnp.float32),
            pltpu.VMEM((2 * C, SH_STRIDE), jnp.float32),
            pltpu.VMEM((ROWS_PER_W,), jnp.float32),
        ],
        compiler_params=pltpu.CompilerParams(use_tc_tiling_on_sc=False,
                                             needs_layout_passes=False),
    )


def _final_body(x_ref, o_ref):
    o_ref[0, 0] = jnp.sum(jnp.sqrt(x_ref[:])) * (1.0 / N)


_finalize = functools.partial(
    pl.pallas_call,
    _final_body,
    in_specs=[pl.BlockSpec((NUM_WORKERS, ROWS_PER_W), lambda: (0, 0))],
    out_specs=pl.BlockSpec(memory_space=pltpu.SMEM),
    out_shape=jax.ShapeDtypeStruct((1, 1), jnp.float32),
)


def kernel(outputs, W, Wh):
    del Wh
    src = outputs[:N]
    dst = outputs[N:2 * N]
    U = outputs[2 * N:]
    Ue = jnp.zeros((C, D), jnp.float32).at[1:1 + M].set(U)
    rd, sh = _components()(Ue, W, src, dst)
    maxd2 = _decode()(rd, sh)
    res = _finalize()(maxd2.reshape(NUM_WORKERS, ROWS_PER_W))
    return res[0, 0]


# speedup vs baseline: 33.5092x; 1.2007x over previous
import functools

import jax
import jax.numpy as jnp
from jax import lax
from jax.experimental import pallas as pl
from jax.experimental.pallas import tpu as pltpu
from jax.experimental.pallas import tpu_sc as plsc

N = 4096
M = 62
D = 128
C = 64
NEG = -1e9

R_BLK = 512


def _comp_body(ue_ref, w_ref, src_ref, dst_ref,
               rest_ref, s63_ref, g63_ref, dc_ref, sh_ref):
    f32 = jnp.float32
    Ue = ue_ref[:]
    W = w_ref[:]
    src = src_ref[:]
    dst = dst_ref[:]

    def mm(a, b):
        return lax.dot_general(a, b, (((1,), (0,)), ((), ())),
                               preferred_element_type=f32)

    def mmt(a, b):
        return lax.dot_general(a, b, (((1,), (1,)), ((), ())),
                               preferred_element_type=f32)

    QUe = mm(Ue, W)
    Qsrc = mm(src, W)
    Qdst = mm(dst, W)

    col = lax.broadcasted_iota(jnp.int32, (R_BLK, C), 1)
    is0 = col == 0
    is63 = col == C - 1

    ndst = jnp.sum(dst * dst, axis=1, keepdims=True)
    nsrc = jnp.sum(src * src, axis=1, keepdims=True)

    S0 = mmt(Qsrc, Ue)
    S63 = mmt(Qdst, Ue)
    S63 = jnp.where(is63, jnp.sum(Qdst * dst, axis=1, keepdims=True), S63)
    DC = mmt(dst, QUe)
    G63 = mmt(dst, Ue)
    G63 = jnp.where(is63, ndst, G63)
    G0 = mmt(src, Ue)
    nU = jnp.sum(Ue * Ue, axis=1)
    NCH = jnp.broadcast_to(nU[None, :], (R_BLK, C))
    NCH = jnp.where(is0, nsrc, NCH)
    NCH = jnp.where(is63, ndst, NCH)

    rest_ref[:] = jnp.concatenate([S0, NCH, G0], axis=1)
    s63_ref[:] = S63
    g63_ref[:] = G63
    dc_ref[:] = DC
    sh_ref[:] = jnp.concatenate([mmt(QUe, Ue), mmt(Ue, Ue)], axis=0)


_components = functools.partial(
    pl.pallas_call,
    _comp_body,
    grid=(N // R_BLK,),
    in_specs=[
        pl.BlockSpec((C, D), lambda i: (0, 0)),
        pl.BlockSpec((D, D), lambda i: (0, 0)),
        pl.BlockSpec((R_BLK, D), lambda i: (i, 0)),
        pl.BlockSpec((R_BLK, D), lambda i: (i, 0)),
    ],
    out_specs=[
        pl.BlockSpec((R_BLK, 3 * C), lambda i: (i, 0)),
        pl.BlockSpec((R_BLK, C), lambda i: (i, 0)),
        pl.BlockSpec((R_BLK, C), lambda i: (i, 0)),
        pl.BlockSpec((R_BLK, C), lambda i: (i, 0)),
        pl.BlockSpec((2 * C, C), lambda i: (0, 0)),
    ],
    out_shape=[
        jax.ShapeDtypeStruct((N, 3 * C), jnp.float32),
        jax.ShapeDtypeStruct((N, C), jnp.float32),
        jax.ShapeDtypeStruct((N, C), jnp.float32),
        jax.ShapeDtypeStruct((N, C), jnp.float32),
        jax.ShapeDtypeStruct((2 * C, C), jnp.float32),
    ],
)


NUM_WORKERS = 32
ROWS_PER_W = N // NUM_WORKERS
GROUPS = ROWS_PER_W // 16
REST_STRIDE = 3 * C + 1
TBL_STRIDE = C + 1
T_S63, T_GUU, T_G63, T_DC = 64, 192, 256, 384
B_N, B_G0 = 64, 128


def _decode_body(rest_hbm, s63_hbm, g63_hbm, dc_hbm, sh_hbm, out_hbm,
                 rest_v, tbl_v, mask_v, out_v):
    wid = lax.axis_index("s") * 2 + lax.axis_index("c")
    base = wid * ROWS_PER_W
    rows = pl.ds(base, ROWS_PER_W)
    cc = pl.ds(0, C)
    pltpu.sync_copy(rest_hbm.at[rows], rest_v.at[:, pl.ds(0, 3 * C)])
    pltpu.sync_copy(sh_hbm.at[pl.ds(0, C)], tbl_v.at[pl.ds(0, C), cc])
    pltpu.sync_copy(s63_hbm.at[rows], tbl_v.at[pl.ds(T_S63, ROWS_PER_W), cc])
    pltpu.sync_copy(sh_hbm.at[pl.ds(C, C)], tbl_v.at[pl.ds(T_GUU, C), cc])
    pltpu.sync_copy(g63_hbm.at[rows], tbl_v.at[pl.ds(T_G63, ROWS_PER_W), cc])
    pltpu.sync_copy(dc_hbm.at[rows], tbl_v.at[pl.ds(T_DC, ROWS_PER_W), cc])

    lanes = lax.iota(jnp.int32, 16)
    zero = jnp.zeros((16,), jnp.int32)
    negv = jnp.full((16,), NEG, jnp.float32)
    zf = jnp.zeros((16,), jnp.float32)
    ninf = jnp.full((16,), -jnp.inf, jnp.float32)

    for g in range(GROUPS):
        rvec = rvec0 = lanes + (g * 16)

        for r in range(16):
            for c4 in range(C // 16):
                mask_v[r, pl.ds(c4 * 16, 16)] = zf

        def gre(b, colv):
            return plsc.load_gather(rest_v, [rvec0, colv + b])

        def s0_body(s, carry):
            best, besti = carry
            sv = zero + s
            v = gre(0, sv)
            gt = v > best
            return jnp.where(gt, v, best), jnp.where(gt, sv, besti)

        _, sp = lax.fori_loop(1, 63, s0_body, (ninf, zero), unroll=16)
        nn = gre(B_N, sp)
        d2 = jnp.maximum(gre(B_N, zero) + nn - 2.0 * gre(B_G0, sp), 0.0)
        maxd2 = d2
        plsc.store_scatter(mask_v, [lanes, sp], negv)

        def step(_, carry):
            j, ncur, md2 = carry
            isD = j == C - 1
            p = jnp.where(isD, T_S63 + rvec, j)

            def inner(s, c):
                best, besti = c
                sv = zero + s
                v = plsc.load_gather(tbl_v, [p, sv])
                m = plsc.load_gather(mask_v, [lanes, sv])
                cand = v + m
                gt = cand > best
                return jnp.where(gt, cand, best), jnp.where(gt, sv, besti)

            best, besti = lax.fori_loop(1, 63, inner, (ninf, zero), unroll=16)
            q = jnp.where(isD, T_S63 + rvec, T_DC + rvec)
            c63 = jnp.where(isD, zero + (C - 1), j)
            v63 = plsc.load_gather(tbl_v, [q, c63])
            gt = v63 > best
            sp = jnp.where(gt, zero + (C - 1), besti)
            nn = gre(B_N, sp)
            sp63 = sp == C - 1
            row2 = jnp.where(isD | sp63, T_G63 + rvec, T_GUU + j)
            col2 = jnp.where(sp63 & (~isD), j, sp)
            gv = plsc.load_gather(tbl_v, [row2, col2])
            d2 = jnp.maximum(ncur + nn - 2.0 * gv, 0.0)
            md2 = jnp.maximum(md2, d2)
            plsc.store_scatter(mask_v, [lanes, sp],
                               jnp.where(sp63, zf, negv))
            return sp, nn, md2

        *_, maxd2 = lax.fori_loop(1, C, step, (sp, nn, maxd2))
        out_v[pl.ds(g * 16, 16)] = maxd2

    pltpu.sync_copy(out_v, out_hbm.at[rows])


@functools.cache
def _decode():
    mesh = plsc.VectorSubcoreMesh(core_axis_name="c", subcore_axis_name="s",
                                  num_cores=2, num_subcores=16)
    return pl.kernel(
        _decode_body,
        out_type=jax.ShapeDtypeStruct((N,), jnp.float32),
        mesh=mesh,
        scratch_types=[
            pltpu.VMEM((ROWS_PER_W, REST_STRIDE), jnp.float32),
            pltpu.VMEM((8 * C, TBL_STRIDE), jnp.float32),
            pltpu.VMEM((16, TBL_STRIDE), jnp.float32),
            pltpu.VMEM((ROWS_PER_W,), jnp.float32),
        ],
        compiler_params=pltpu.CompilerParams(use_tc_tiling_on_sc=False,
                                             needs_layout_passes=False),
    )


def _final_body(x_ref, o_ref):
    o_ref[0, 0] = jnp.sum(jnp.sqrt(x_ref[:])) * (1.0 / N)


_finalize = functools.partial(
    pl.pallas_call,
    _final_body,
    in_specs=[pl.BlockSpec((NUM_WORKERS, ROWS_PER_W), lambda: (0, 0))],
    out_specs=pl.BlockSpec(memory_space=pltpu.SMEM),
    out_shape=jax.ShapeDtypeStruct((1, 1), jnp.float32),
)


def kernel(outputs, W, Wh):
    del Wh
    src = outputs[:N]
    dst = outputs[N:2 * N]
    U = outputs[2 * N:]
    Ue = jnp.zeros((C, D), jnp.float32).at[1:1 + M].set(U)
    rest, s63, g63, dc, sh = _components()(Ue, W, src, dst)
    maxd2 = _decode()(rest, s63, g63, dc, sh)
    res = _finalize()(maxd2.reshape(NUM_WORKERS, ROWS_PER_W))
    return res[0, 0]
